# Initial kernel scaffold; baseline (speedup 1.0000x reference)
import jax
import jax.numpy as jnp
from jax.experimental import pallas as pl


def kernel(h, edge_index, frac_cover_mat, W1, a1, W2, a2, Wf1, Wf2, Wf3, Wff, Wfc2):
    raise NotImplementedError("write your pallas kernel here")


# speedup vs baseline: 9.4220x; 9.4220x over previous
import functools

import jax
import jax.numpy as jnp
from jax import lax
from jax.experimental import pallas as pl
from jax.experimental.pallas import tpu as pltpu
from jax.experimental.pallas import tpu_sc as plsc

N = 10000
E = 320000
D = 128
H = 128
OUT = 128

F32 = jnp.float32
I32 = jnp.int32

NC = 2
NS = 16
NW = NC * NS

AW = H + 16
EPW = E // NW
EC = 80
NCHUNK = EPW // EC
NPAD = 10240
ROWS_PT = NPAD // NS
ZROWS = 128

GB = 30720
GW = GB // NW
GC = 120
GCHUNK = GW // GC

_sc_mesh = plsc.VectorSubcoreMesh(core_axis_name="c", subcore_axis_name="s")


@functools.partial(
    pl.kernel,
    out_type=jax.ShapeDtypeStruct((NC, NPAD, AW), F32),
    mesh=_sc_mesh,
    compiler_params=pltpu.CompilerParams(use_tc_tiling_on_sc=False, needs_layout_passes=False),
    scratch_types=[
        pltpu.VMEM_SHARED((NPAD, AW), F32),
        pltpu.VMEM((N,), F32),
        pltpu.VMEM((EC,), I32),
        pltpu.VMEM((EC,), I32),
        pltpu.VMEM((EC,), F32),
        pltpu.VMEM((EC, AW), F32),
        pltpu.VMEM((EC, AW), F32),
        pltpu.VMEM((16,), F32),
        pltpu.SemaphoreType.DMA,
    ],
)
def _edge_stage(z_hbm, ei_hbm, s_hbm, b_hbm, acc_out,
                acc_sh, sb_v, src_v, dst_v, ex_v, rows_v, comb_v,
                bvec_v, sem):
    c = lax.axis_index("c")
    s = lax.axis_index("s")
    wid = s * NC + c
    zero16 = jnp.zeros((16,), F32)

    def _crow(r, carry):
        for k in range(AW // 16):
            comb_v[r, pl.ds(k * 16, 16)] = zero16
        return carry
    lax.fori_loop(0, EC, _crow, 0)

    row0 = s * ROWS_PT
    for i in range(ROWS_PT // EC):
        pltpu.sync_copy(comb_v, acc_sh.at[pl.ds(row0 + i * EC, EC)])

    pltpu.sync_copy(s_hbm.at[pl.ds(N, N)], sb_v)
    pltpu.sync_copy(b_hbm, bvec_v)
    plsc.subcore_barrier()

    bv = bvec_v[...]
    ebase = wid * EPW
    col_sa = jnp.full((16,), D, I32)

    def _chunk(j, carry):
        off = ebase + j * EC
        pltpu.sync_copy(ei_hbm.at[pl.ds(off, EC)], src_v)
        pltpu.sync_copy(ei_hbm.at[pl.ds(E + off, EC)], dst_v)
        pltpu.async_copy(z_hbm.at[src_v], rows_v, sem).wait()
        for g in range(EC // 16):
            rid = lax.iota(I32, 16) + g * 16
            di = dst_v[pl.ds(g * 16, 16)]
            e = (plsc.load_gather(rows_v, [rid, col_sa])
                 + plsc.load_gather(sb_v, [di]))
            e = jnp.where(e > 0.0, e, e * 0.01)
            ex = jnp.exp(e - bv)
            ex_v[pl.ds(g * 16, 16)] = ex
            plsc.store_scatter(comb_v, [rid, col_sa], ex)

        def _scale(r, carry2):
            exs = plsc.load_gather(ex_v, [jnp.broadcast_to(r, (16,))])
            for k in range(D // 16):
                comb_v[r, pl.ds(k * 16, 16)] = rows_v[r, pl.ds(k * 16, 16)] * exs
            return carry2
        lax.fori_loop(0, EC, _scale, 0)

        pltpu.sync_copy(comb_v, acc_sh.at[dst_v], add=True)
        return carry

    lax.fori_loop(0, NCHUNK, _chunk, 0)
    plsc.subcore_barrier()

    for i in range(ROWS_PT // ZROWS):
        r = row0 + i * ZROWS
        pltpu.sync_copy(acc_sh.at[pl.ds(r, ZROWS)], acc_out.at[c, pl.ds(r, ZROWS)])


@functools.partial(
    pl.kernel,
    out_type=jax.ShapeDtypeStruct((GB, D), F32),
    mesh=_sc_mesh,
    compiler_params=pltpu.CompilerParams(use_tc_tiling_on_sc=False, needs_layout_passes=False),
    scratch_types=[
        pltpu.VMEM((GC,), I32),
        pltpu.VMEM((GC, D), F32),
        pltpu.SemaphoreType.DMA,
    ],
)
def _frac_gather(h_hbm, idx_hbm, out_hbm, idx_v, rows_v, sem):
    c = lax.axis_index("c")
    s = lax.axis_index("s")
    base = (s * NC + c) * GW

    def _chunk(j, carry):
        off = base + j * GC
        pltpu.sync_copy(idx_hbm.at[pl.ds(off, GC)], idx_v)
        pltpu.async_copy(h_hbm.at[idx_v], rows_v, sem).wait()
        pltpu.sync_copy(rows_v, out_hbm.at[pl.ds(off, GC)])
        return carry

    lax.fori_loop(0, GCHUNK, _chunk, 0)


def _store_z_ext(z, a_ref, z_ref, s_ref, b_ref):
    a2 = jnp.concatenate([a_ref[0:1, :H], a_ref[0:1, H:]], axis=0)
    s2 = lax.dot_general(a2, z, (((1,), (1,)), ((), ())),
                         preferred_element_type=F32)
    s_ref[...] = s2
    sa_col = jnp.sum(z * a_ref[0:1, :H], axis=1, keepdims=True)
    z_ref[...] = jnp.concatenate(
        [z, sa_col, jnp.zeros((N, AW - D - 1), F32)], axis=1)
    bnd = jnp.maximum(jnp.max(s2[0:1, :]) + jnp.max(s2[1:2, :]), 0.0)
    b_ref[...] = jnp.broadcast_to(bnd, (1, 16))


def _prep_body(h_ref, w_ref, a_ref, z_ref, s_ref, b_ref):
    z = lax.dot_general(h_ref[...], w_ref[...],
                        (((1,), (1,)), ((), ())), preferred_element_type=F32)
    _store_z_ext(z, a_ref, z_ref, s_ref, b_ref)


_prep = pl.pallas_call(
    _prep_body,
    out_shape=(
        jax.ShapeDtypeStruct((N, AW), F32),
        jax.ShapeDtypeStruct((2, N), F32),
        jax.ShapeDtypeStruct((1, 16), F32),
    ),
)


def _mid_body(acc_ref, w_ref, a_ref, z_ref, s_ref, b_ref):
    accs = acc_ref[0, :N] + acc_ref[1, :N]
    o = accs[:, :H] / jnp.maximum(accs[:, H:H + 1], 1e-16)
    h1 = jnp.where(o > 0.0, o, jnp.exp(o) - 1.0)
    z = lax.dot_general(h1, w_ref[...], (((1,), (1,)), ((), ())),
                        preferred_element_type=F32)
    _store_z_ext(z, a_ref, z_ref, s_ref, b_ref)


_mid = pl.pallas_call(
    _mid_body,
    out_shape=(
        jax.ShapeDtypeStruct((N, AW), F32),
        jax.ShapeDtypeStruct((2, N), F32),
        jax.ShapeDtypeStruct((1, 16), F32),
    ),
)


def _wprep_body(wff_ref, wfc2_ref, wf1_ref, wf2_ref, wf3_ref, c_ref):
    b = wfc2_ref[:, H:]
    for k, wf in enumerate((wf1_ref, wf2_ref, wf3_ref)):
        m = lax.dot_general(b, wff_ref[:, k * H:(k + 1) * H],
                            (((1,), (0,)), ((), ())), preferred_element_type=F32)
        c_ref[k] = lax.dot_general(m, wf[...], (((1,), (0,)), ((), ())),
                                   preferred_element_type=F32)


_wprep = pl.pallas_call(
    _wprep_body,
    out_shape=jax.ShapeDtypeStruct((3, OUT, D), F32),
)


def _final_body(acc_ref, g_ref, wa_ref, c_ref, out_ref):
    accs = acc_ref[0, :N] + acc_ref[1, :N]
    h2 = accs[:, :H] / jnp.maximum(accs[:, H:H + 1], 1e-16)
    out = lax.dot_general(h2, wa_ref[...], (((1,), (1,)), ((), ())),
                          preferred_element_type=F32)
    for k in range(3):
        out = out + lax.dot_general(g_ref[k], c_ref[k],
                                    (((1,), (1,)), ((), ())),
                                    preferred_element_type=F32)
    out_ref[...] = out


_final = pl.pallas_call(
    _final_body,
    out_shape=jax.ShapeDtypeStruct((N, OUT), F32),
)


def kernel(h, edge_index, frac_cover_mat, W1, a1, W2, a2, Wf1, Wf2, Wf3, Wff, Wfc2):
    cmats = _wprep(Wff, Wfc2, Wf1, Wf2, Wf3)
    wa = Wfc2[:, :H]

    idxf = jnp.concatenate(
        [frac_cover_mat.T.reshape(-1).astype(I32),
         jnp.zeros((GB - 3 * N,), I32)])
    g = _frac_gather(h, idxf)
    g3 = g[:3 * N].reshape(3, N, D)

    eiflat = edge_index.reshape(-1)
    z1, s1, b1 = _prep(h, W1, a1)
    acc1 = _edge_stage(z1, eiflat, s1.reshape(-1), b1.reshape(-1))
    z2, s2, b2 = _mid(acc1, W2, a2)
    acc2 = _edge_stage(z2, eiflat, s2.reshape(-1), b2.reshape(-1))

    return _final(acc2, g3, wa, cmats)
